# Initial kernel scaffold; baseline (speedup 1.0000x reference)
import functools

import jax
import jax.numpy as jnp
from jax import lax
from jax.experimental import pallas as pl
from jax.experimental.pallas import tpu as pltpu
from jax.experimental.pallas import tpu_sc as plsc

_VOCAB = 100000
_EMB = 64
_NCLS = 4
_B = 4096
_N = 204800

_NC = 2
_NS = 16
_NW = _NC * _NS
_CHUNK = _N // _NW
_SUB = 128
_NSUB = _CHUNK // _SUB
_BPW = _B // _NW
_BPT = _B // _NS


def _sc_body(text_h, off_h, tab_h, s_h, cnt_h,
             off_v, text2, seg2, rows_a, rows_b, cnt_v, acc_sh, gsem, tsem):
    cid = lax.axis_index("c")
    sid = lax.axis_index("s")
    wid = sid * _NC + cid
    s0 = wid * _CHUNK

    pltpu.sync_copy(off_h, off_v.at[pl.ds(0, _B)])
    off_v[pl.ds(_B, 16)] = jnp.full((16,), _N, jnp.int32)

    tdesc = [
        pltpu.async_copy(text_h.at[pl.ds(s0 + j * _SUB, _SUB)], text2.at[j], tsem)
        for j in range(_NSUB)
    ]

    def _zseg(k, c):
        seg2[k >> 3, pl.ds((k & 7) * 16, 16)] = jnp.zeros((16,), jnp.int32)
        return c
    lax.fori_loop(0, _NSUB * 8, _zseg, 0)

    ones16 = jnp.ones((16,), jnp.int32)

    def _mark(i, acc):
        ov = off_v[pl.ds(i * 16, 16)]
        p = ov - s0
        m = (p >= 0) & (p < _CHUNK)
        plsc.store_scatter(seg2, [p >> 7, p & 127], ones16, mask=m)
        return acc + jnp.where(ov < s0, 1, 0)

    acc = lax.fori_loop(0, _B // 16, _mark, jnp.zeros((16,), jnp.int32))
    base = jnp.sum(acc)

    def _cum(k, carry):
        r = k >> 3
        c = (k & 7) * 16
        v = seg2[r, pl.ds(c, 16)]
        seg2[r, pl.ds(c, 16)] = jnp.cumsum(v) + carry
        return carry + jnp.sum(v)

    lax.fori_loop(0, _NSUB * 8, _cum, base - 1)

    bbase = wid * _BPW
    for k in range(_BPW // 16):
        a = off_v[pl.ds(bbase + k * 16, 16)]
        nx = off_v[pl.ds(bbase + k * 16 + 1, 16)]
        cnt_v[pl.ds(k * 16, 16)] = (nx - a).astype(jnp.float32)
    pltpu.sync_copy(cnt_v, cnt_h.at[pl.ds(bbase, _BPW)])

    def _zrows(k, c):
        rows_a[k >> 2, pl.ds((k & 3) * 16, 16)] = jnp.zeros((16,), jnp.float32)
        return c
    lax.fori_loop(0, _SUB * 4, _zrows, 0)
    pltpu.sync_copy(rows_a, acc_sh.at[pl.ds(sid * _BPT, _SUB)])
    pltpu.sync_copy(rows_a, acc_sh.at[pl.ds(sid * _BPT + _SUB, _SUB)])
    plsc.subcore_barrier()

    for d in tdesc:
        d.wait()

    def _main(j, c):
        g = pltpu.async_copy(tab_h.at[text2.at[j]], rows_a, gsem)
        g.wait()
        pltpu.sync_copy(rows_a, acc_sh.at[seg2.at[j]], add=True)
        return c

    lax.fori_loop(0, _NSUB, _main, 0)
    plsc.subcore_barrier()

    pltpu.sync_copy(acc_sh.at[pl.ds(sid * _BPT, _SUB)], rows_a)
    pltpu.sync_copy(rows_a, s_h.at[cid, pl.ds(sid * _BPT, _SUB)])
    pltpu.sync_copy(acc_sh.at[pl.ds(sid * _BPT + _SUB, _SUB)], rows_b)
    pltpu.sync_copy(rows_b, s_h.at[cid, pl.ds(sid * _BPT + _SUB, _SUB)])


_sc_call = functools.partial(
    pl.kernel,
    out_type=[
        jax.ShapeDtypeStruct((_NC, _B, _EMB), jnp.float32),
        jax.ShapeDtypeStruct((_B,), jnp.float32),
    ],
    mesh=plsc.VectorSubcoreMesh(core_axis_name="c", subcore_axis_name="s"),
    scratch_types=[
        pltpu.VMEM((_B + 16,), jnp.int32),
        pltpu.VMEM((_NSUB, _SUB), jnp.int32),
        pltpu.VMEM((_NSUB, _SUB), jnp.int32),
        pltpu.VMEM((_SUB, _EMB), jnp.float32),
        pltpu.VMEM((_SUB, _EMB), jnp.float32),
        pltpu.VMEM((_BPW,), jnp.float32),
        pltpu.VMEM_SHARED((_B, _EMB), jnp.float32),
        pltpu.SemaphoreType.DMA,
        pltpu.SemaphoreType.DMA,
    ],
)(_sc_body)


def _tc_body(s0_ref, s1_ref, cnt_ref, w_ref, b_ref, o_ref):
    s = s0_ref[...] + s1_ref[...]
    pooled = s / cnt_ref[...]
    o_ref[...] = (
        lax.dot_general(pooled, w_ref[...], (((1,), (1,)), ((), ())),
                        preferred_element_type=jnp.float32)
        + b_ref[...]
    )


_tc_call = pl.pallas_call(
    _tc_body,
    out_shape=jax.ShapeDtypeStruct((_B, _NCLS), jnp.float32),
)


def kernel(text, offsets, emb_table, W, b):
    s_parts, cnt = _sc_call(text, offsets, emb_table)
    return _tc_call(s_parts[0], s_parts[1], cnt.reshape(_B, 1),
                    W, b.reshape(1, _NCLS))


# speedup vs baseline: 124.5140x; 124.5140x over previous
import functools

import jax
import jax.numpy as jnp
from jax import lax
from jax.experimental import pallas as pl
from jax.experimental.pallas import tpu as pltpu
from jax.experimental.pallas import tpu_sc as plsc

_VOCAB = 100000
_EMB = 64
_NCLS = 4
_B = 4096
_N = 204800

_NC = 2
_NS = 16
_NW = _NC * _NS
_CHUNK = _N // _NW
_SUB = 128
_NSUB = _CHUNK // _SUB
_BPW = _B // _NW
_BPT = _B // _NS


def _sc_body(text_h, off_h, tab_h, s_h, cnt_h,
             off_v, text1, seg1, idx128, rows_a, rows_b, cnt_v, acc_sh,
             gsem, tsem):
    cid = lax.axis_index("c")
    sid = lax.axis_index("s")
    wid = sid * _NC + cid
    s0 = wid * _CHUNK

    pltpu.sync_copy(off_h, off_v.at[pl.ds(0, _B)])
    off_v[pl.ds(_B, 16)] = jnp.full((16,), _N, jnp.int32)

    tdesc = pltpu.async_copy(text_h.at[pl.ds(s0, _CHUNK)], text1, tsem)

    def _zseg(k, c):
        seg1[pl.ds(k * 16, 16)] = jnp.zeros((16,), jnp.int32)
        return c
    lax.fori_loop(0, _CHUNK // 16, _zseg, 0)

    ones16 = jnp.ones((16,), jnp.int32)

    def _mark(i, acc):
        ov = off_v[pl.ds(i * 16, 16)]
        p = ov - s0
        m = (p >= 0) & (p < _CHUNK)
        plsc.store_scatter(seg1, [p], ones16, mask=m)
        return acc + jnp.where(ov < s0, 1, 0)

    acc = lax.fori_loop(0, _B // 16, _mark, jnp.zeros((16,), jnp.int32))
    base = jnp.sum(acc)

    def _cum(k, carry):
        v = seg1[pl.ds(k * 16, 16)]
        seg1[pl.ds(k * 16, 16)] = jnp.cumsum(v) + carry
        return carry + jnp.sum(v)

    lax.fori_loop(0, _CHUNK // 16, _cum, base - 1)

    bbase = wid * _BPW
    for k in range(_BPW // 16):
        a = off_v[pl.ds(bbase + k * 16, 16)]
        nx = off_v[pl.ds(bbase + k * 16 + 1, 16)]
        cnt_v[pl.ds(k * 16, 16)] = (nx - a).astype(jnp.float32)
    pltpu.sync_copy(cnt_v, cnt_h.at[pl.ds(bbase, _BPW)])

    def _zrows(k, c):
        rows_a[k >> 2, pl.ds((k & 3) * 16, 16)] = jnp.zeros((16,), jnp.float32)
        return c
    lax.fori_loop(0, _SUB * 4, _zrows, 0)
    pltpu.sync_copy(rows_a, acc_sh.at[pl.ds(sid * _BPT, _SUB)])
    pltpu.sync_copy(rows_a, acc_sh.at[pl.ds(sid * _BPT + _SUB, _SUB)])
    plsc.subcore_barrier()

    tdesc.wait()

    def _main(j, c):
        g = pltpu.async_copy(tab_h.at[text1.at[pl.ds(j * _SUB, _SUB)]],
                             rows_a, gsem)
        for t in range(_SUB // 16):
            idx128[pl.ds(t * 16, 16)] = seg1[pl.ds(j * _SUB + t * 16, 16)]
        g.wait()
        pltpu.sync_copy(rows_a, acc_sh.at[idx128], add=True)
        return c

    lax.fori_loop(0, _NSUB, _main, 0)
    plsc.subcore_barrier()

    pltpu.sync_copy(acc_sh.at[pl.ds(sid * _BPT, _SUB)], rows_a)
    pltpu.sync_copy(rows_a, s_h.at[cid, pl.ds(sid * _BPT, _SUB)])
    pltpu.sync_copy(acc_sh.at[pl.ds(sid * _BPT + _SUB, _SUB)], rows_b)
    pltpu.sync_copy(rows_b, s_h.at[cid, pl.ds(sid * _BPT + _SUB, _SUB)])


_sc_call = functools.partial(
    pl.kernel,
    out_type=[
        jax.ShapeDtypeStruct((_NC, _B, _EMB), jnp.float32),
        jax.ShapeDtypeStruct((_B,), jnp.float32),
    ],
    mesh=plsc.VectorSubcoreMesh(core_axis_name="c", subcore_axis_name="s"),
    compiler_params=pltpu.CompilerParams(needs_layout_passes=False,
                                         use_tc_tiling_on_sc=False),
    scratch_types=[
        pltpu.VMEM((_B + 16,), jnp.int32),
        pltpu.VMEM((_CHUNK,), jnp.int32),
        pltpu.VMEM((_CHUNK,), jnp.int32),
        pltpu.VMEM((_SUB,), jnp.int32),
        pltpu.VMEM((_SUB, _EMB), jnp.float32),
        pltpu.VMEM((_SUB, _EMB), jnp.float32),
        pltpu.VMEM((_BPW,), jnp.float32),
        pltpu.VMEM_SHARED((_B, _EMB), jnp.float32),
        pltpu.SemaphoreType.DMA,
        pltpu.SemaphoreType.DMA,
    ],
)(_sc_body)


def _tc_body(s0_ref, s1_ref, cnt_ref, w_ref, b_ref, o_ref):
    s = s0_ref[...] + s1_ref[...]
    pooled = s / cnt_ref[...]
    o_ref[...] = (
        lax.dot_general(pooled, w_ref[...], (((1,), (1,)), ((), ())),
                        preferred_element_type=jnp.float32)
        + b_ref[...]
    )


_tc_call = pl.pallas_call(
    _tc_body,
    out_shape=jax.ShapeDtypeStruct((_B, _NCLS), jnp.float32),
)


def kernel(text, offsets, emb_table, W, b):
    s_parts, cnt = _sc_call(text, offsets, emb_table)
    return _tc_call(s_parts[0], s_parts[1], cnt.reshape(_B, 1),
                    W, b.reshape(1, _NCLS))


# speedup vs baseline: 146.8174x; 1.1791x over previous
import functools

import jax
import jax.numpy as jnp
from jax import lax
from jax.experimental import pallas as pl
from jax.experimental.pallas import tpu as pltpu
from jax.experimental.pallas import tpu_sc as plsc

_VOCAB = 100000
_EMB = 64
_NCLS = 4
_B = 4096
_N = 204800

_NC = 2
_NS = 16
_NW = _NC * _NS
_CHUNK = _N // _NW
_SUB = 128
_NSUB = _CHUNK // _SUB
_BPT = _B // _NS


def _sc_body(text_h, off_h, tab_h, s_h,
             off_v, text1, seg1, rows_a, rows_b, inv_v, acc_sh,
             gsa, gsb, ssa, ssb, osem):
    cid = lax.axis_index("c")
    sid = lax.axis_index("s")
    wid = sid * _NC + cid
    s0 = wid * _CHUNK

    odesc = pltpu.async_copy(off_h, off_v.at[pl.ds(0, _B)], osem)
    tdesc = pltpu.async_copy(text_h.at[pl.ds(s0, _CHUNK)], text1, gsa)

    z16 = jnp.zeros((16,), jnp.int32)

    def _zseg(k, c):
        for t in range(4):
            seg1[pl.ds(k * 64 + t * 16, 16)] = z16
        return c
    lax.fori_loop(0, _CHUNK // 64, _zseg, 0)

    zf16 = jnp.zeros((16,), jnp.float32)

    def _zrows(k, c):
        for t in range(4):
            rows_a[k, pl.ds(t * 16, 16)] = zf16
        return c
    lax.fori_loop(0, _SUB, _zrows, 0)
    pltpu.sync_copy(rows_a, acc_sh.at[pl.ds(sid * _BPT, _SUB)])
    pltpu.sync_copy(rows_a, acc_sh.at[pl.ds(sid * _BPT + _SUB, _SUB)])

    odesc.wait()
    off_v[pl.ds(_B, 16)] = jnp.full((16,), _N, jnp.int32)

    ones16 = jnp.ones((16,), jnp.int32)

    def _mark(i, acc):
        a = acc
        for t in range(4):
            ov = off_v[pl.ds(i * 64 + t * 16, 16)]
            p = ov - s0
            m = (p >= 0) & (p < _CHUNK)
            plsc.store_scatter(seg1, [p], ones16, mask=m)
            a = a + jnp.where(ov < s0, 1, 0)
        return a

    acc = lax.fori_loop(0, _B // 64, _mark, jnp.zeros((16,), jnp.int32))
    base = jnp.sum(acc)

    def _cum(k, carry):
        vs = [seg1[pl.ds(k * 64 + t * 16, 16)] for t in range(4)]
        css = [jnp.cumsum(v) for v in vs]
        sums = [jnp.sum(v) for v in vs]
        c = carry
        for t in range(4):
            seg1[pl.ds(k * 64 + t * 16, 16)] = css[t] + c
            c = c + sums[t]
        return c

    lax.fori_loop(0, _CHUNK // 64, _cum, base - 1)

    plsc.subcore_barrier()
    tdesc.wait()

    def _fire_gather(j, buf, sem):
        return pltpu.async_copy(tab_h.at[text1.at[pl.ds(j * _SUB, _SUB)]],
                                buf, sem)

    def _fire_scatter(j, buf, sem):
        return pltpu.async_copy(buf, acc_sh.at[seg1.at[pl.ds(j * _SUB, _SUB)]],
                                sem, add=True)

    _fire_gather(0, rows_a, gsa)
    _fire_gather(1, rows_b, gsb)

    def _main(g, c):
        j0 = g * 2
        pltpu.make_async_copy(tab_h.at[text1.at[pl.ds(j0 * _SUB, _SUB)]],
                              rows_a, gsa).wait()
        sa = _fire_scatter(j0, rows_a, ssa)
        pltpu.make_async_copy(tab_h.at[text1.at[pl.ds((j0 + 1) * _SUB, _SUB)]],
                              rows_b, gsb).wait()
        sb = _fire_scatter(j0 + 1, rows_b, ssb)
        sa.wait()
        _fire_gather(j0 + 2, rows_a, gsa)
        sb.wait()
        _fire_gather(j0 + 3, rows_b, gsb)
        return c

    lax.fori_loop(0, _NSUB // 2 - 1, _main, 0)

    jl = _NSUB - 2
    pltpu.make_async_copy(tab_h.at[text1.at[pl.ds(jl * _SUB, _SUB)]],
                          rows_a, gsa).wait()
    sa = _fire_scatter(jl, rows_a, ssa)
    pltpu.make_async_copy(tab_h.at[text1.at[pl.ds((jl + 1) * _SUB, _SUB)]],
                          rows_b, gsb).wait()
    sb = _fire_scatter(jl + 1, rows_b, ssb)
    sa.wait()
    sb.wait()
    plsc.subcore_barrier()

    tb = sid * _BPT
    for k in range(_BPT // 16):
        a = off_v[pl.ds(tb + k * 16, 16)]
        nx = off_v[pl.ds(tb + k * 16 + 1, 16)]
        inv_v[pl.ds(k * 16, 16)] = 1.0 / (nx - a).astype(jnp.float32)

    def _scale(buf, half):
        def _row(r, c):
            gv = plsc.load_gather(inv_v, [jnp.full((16,), half * _SUB + r,
                                                   jnp.int32)])
            for t in range(4):
                v = buf[r, pl.ds(t * 16, 16)]
                buf[r, pl.ds(t * 16, 16)] = v * gv
            return c
        lax.fori_loop(0, _SUB, _row, 0)

    pltpu.sync_copy(acc_sh.at[pl.ds(tb, _SUB)], rows_a)
    _scale(rows_a, 0)
    wa = pltpu.async_copy(rows_a, s_h.at[cid, pl.ds(tb, _SUB)], osem)
    pltpu.sync_copy(acc_sh.at[pl.ds(tb + _SUB, _SUB)], rows_b)
    _scale(rows_b, 1)
    wa.wait()
    pltpu.sync_copy(rows_b, s_h.at[cid, pl.ds(tb + _SUB, _SUB)])


_sc_call = functools.partial(
    pl.kernel,
    out_type=[
        jax.ShapeDtypeStruct((_NC, _B, _EMB), jnp.float32),
    ],
    mesh=plsc.VectorSubcoreMesh(core_axis_name="c", subcore_axis_name="s"),
    compiler_params=pltpu.CompilerParams(needs_layout_passes=False,
                                         use_tc_tiling_on_sc=False),
    scratch_types=[
        pltpu.VMEM((_B + 16,), jnp.int32),
        pltpu.VMEM((_CHUNK,), jnp.int32),
        pltpu.VMEM((_CHUNK,), jnp.int32),
        pltpu.VMEM((_SUB, _EMB), jnp.float32),
        pltpu.VMEM((_SUB, _EMB), jnp.float32),
        pltpu.VMEM((_BPT,), jnp.float32),
        pltpu.VMEM_SHARED((_B, _EMB), jnp.float32),
        pltpu.SemaphoreType.DMA,
        pltpu.SemaphoreType.DMA,
        pltpu.SemaphoreType.DMA,
        pltpu.SemaphoreType.DMA,
        pltpu.SemaphoreType.DMA,
    ],
)(_sc_body)


def _tc_body(s_ref, w_ref, b_ref, o_ref):
    pooled = s_ref[0] + s_ref[1]
    o_ref[...] = (
        lax.dot_general(pooled, w_ref[...], (((1,), (1,)), ((), ())),
                        preferred_element_type=jnp.float32)
        + b_ref[...]
    )


_tc_call = pl.pallas_call(
    _tc_body,
    out_shape=jax.ShapeDtypeStruct((_B, _NCLS), jnp.float32),
)


def kernel(text, offsets, emb_table, W, b):
    (s_parts,) = _sc_call(text, offsets, emb_table)
    return _tc_call(s_parts, W, b.reshape(1, _NCLS))


# speedup vs baseline: 150.5739x; 1.0256x over previous
import functools

import jax
import jax.numpy as jnp
from jax import lax
from jax.experimental import pallas as pl
from jax.experimental.pallas import tpu as pltpu
from jax.experimental.pallas import tpu_sc as plsc
from jax.experimental.layout import (Format as _Format, Layout as _Layout,
                                     with_layout_constraint
                                     as _with_layout_constraint)

_VOCAB = 100000
_EMB = 64
_NCLS = 4
_B = 4096
_N = 204800

_NC = 2
_NS = 16
_NW = _NC * _NS
_CHUNK = _N // _NW
_SUB = 128
_NSUB = _CHUNK // _SUB
_BPT = _B // _NS


def _sc_body(text_h, off_h, tab_h, s_h,
             off_v, text1, seg1, rows_a, rows_b, inv_v, acc_sh,
             gsa, gsb, ssa, ssb, osem):
    cid = lax.axis_index("c")
    sid = lax.axis_index("s")
    wid = sid * _NC + cid
    s0 = wid * _CHUNK

    odesc = pltpu.async_copy(off_h, off_v.at[pl.ds(0, _B)], osem)
    tdesc = pltpu.async_copy(text_h.at[pl.ds(s0, _CHUNK)], text1, gsa)

    z16 = jnp.zeros((16,), jnp.int32)

    def _zseg(k, c):
        for t in range(4):
            seg1[pl.ds(k * 64 + t * 16, 16)] = z16
        return c
    lax.fori_loop(0, _CHUNK // 64, _zseg, 0)

    zf16 = jnp.zeros((16,), jnp.float32)

    def _zrows(k, c):
        for t in range(4):
            rows_a[k, pl.ds(t * 16, 16)] = zf16
        return c
    lax.fori_loop(0, _SUB, _zrows, 0)
    pltpu.sync_copy(rows_a, acc_sh.at[pl.ds(sid * _BPT, _SUB)])
    pltpu.sync_copy(rows_a, acc_sh.at[pl.ds(sid * _BPT + _SUB, _SUB)])

    odesc.wait()
    off_v[pl.ds(_B, 16)] = jnp.full((16,), _N, jnp.int32)

    ones16 = jnp.ones((16,), jnp.int32)

    def _mark(i, acc):
        a = acc
        for t in range(4):
            ov = off_v[pl.ds(i * 64 + t * 16, 16)]
            p = ov - s0
            m = (p >= 0) & (p < _CHUNK)
            plsc.store_scatter(seg1, [p], ones16, mask=m)
            a = a + jnp.where(ov < s0, 1, 0)
        return a

    acc = lax.fori_loop(0, _B // 64, _mark, jnp.zeros((16,), jnp.int32))
    base = jnp.sum(acc)

    def _cum(k, carry):
        vs = [seg1[pl.ds(k * 64 + t * 16, 16)] for t in range(4)]
        css = [jnp.cumsum(v) for v in vs]
        sums = [jnp.sum(v) for v in vs]
        c = carry
        for t in range(4):
            seg1[pl.ds(k * 64 + t * 16, 16)] = css[t] + c
            c = c + sums[t]
        return c

    lax.fori_loop(0, _CHUNK // 64, _cum, base - 1)

    plsc.subcore_barrier()
    tdesc.wait()

    tab2 = tab_h

    def _fire_gather(j, buf, sem):
        return pltpu.async_copy(tab2.at[text1.at[pl.ds(j * _SUB, _SUB)]],
                                buf, sem)

    def _fire_scatter(j, buf, sem):
        return pltpu.async_copy(buf, acc_sh.at[seg1.at[pl.ds(j * _SUB, _SUB)]],
                                sem, add=True)

    _fire_gather(0, rows_a, gsa)
    _fire_gather(1, rows_b, gsb)

    def _main(g, c):
        j0 = g * 2
        pltpu.make_async_copy(tab2.at[text1.at[pl.ds(j0 * _SUB, _SUB)]],
                              rows_a, gsa).wait()
        sa = _fire_scatter(j0, rows_a, ssa)
        pltpu.make_async_copy(tab2.at[text1.at[pl.ds((j0 + 1) * _SUB, _SUB)]],
                              rows_b, gsb).wait()
        sb = _fire_scatter(j0 + 1, rows_b, ssb)
        sa.wait()
        _fire_gather(j0 + 2, rows_a, gsa)
        sb.wait()
        _fire_gather(j0 + 3, rows_b, gsb)
        return c

    lax.fori_loop(0, _NSUB // 2 - 1, _main, 0)

    jl = _NSUB - 2
    pltpu.make_async_copy(tab2.at[text1.at[pl.ds(jl * _SUB, _SUB)]],
                          rows_a, gsa).wait()
    sa = _fire_scatter(jl, rows_a, ssa)
    pltpu.make_async_copy(tab2.at[text1.at[pl.ds((jl + 1) * _SUB, _SUB)]],
                          rows_b, gsb).wait()
    sb = _fire_scatter(jl + 1, rows_b, ssb)
    sa.wait()
    sb.wait()
    plsc.subcore_barrier()

    tb = sid * _BPT
    for k in range(_BPT // 16):
        a = off_v[pl.ds(tb + k * 16, 16)]
        nx = off_v[pl.ds(tb + k * 16 + 1, 16)]
        inv_v[pl.ds(k * 16, 16)] = 1.0 / (nx - a).astype(jnp.float32)

    def _scale(buf, half):
        def _row(r, c):
            gv = plsc.load_gather(inv_v, [jnp.full((16,), half * _SUB + r,
                                                   jnp.int32)])
            for t in range(4):
                v = buf[r, pl.ds(t * 16, 16)]
                buf[r, pl.ds(t * 16, 16)] = v * gv
            return c
        lax.fori_loop(0, _SUB, _row, 0)

    pltpu.sync_copy(acc_sh.at[pl.ds(tb, _SUB)], rows_a)
    _scale(rows_a, 0)
    wa = pltpu.async_copy(rows_a, s_h.at[cid, pl.ds(tb, _SUB)], osem)
    pltpu.sync_copy(acc_sh.at[pl.ds(tb + _SUB, _SUB)], rows_b)
    _scale(rows_b, 1)
    wa.wait()
    pltpu.sync_copy(rows_b, s_h.at[cid, pl.ds(tb + _SUB, _SUB)])


_sc_call = functools.partial(
    pl.kernel,
    out_type=[
        jax.ShapeDtypeStruct((_NC, _B, _EMB), jnp.float32),
    ],
    mesh=plsc.VectorSubcoreMesh(core_axis_name="c", subcore_axis_name="s"),
    compiler_params=pltpu.CompilerParams(needs_layout_passes=False,
                                         use_tc_tiling_on_sc=False),
    scratch_types=[
        pltpu.VMEM((_B + 16,), jnp.int32),
        pltpu.VMEM((_CHUNK,), jnp.int32),
        pltpu.VMEM((_CHUNK,), jnp.int32),
        pltpu.VMEM((_SUB, _EMB), jnp.float32),
        pltpu.VMEM((_SUB, _EMB), jnp.float32),
        pltpu.VMEM((_BPT,), jnp.float32),
        pltpu.VMEM_SHARED((_B, _EMB), jnp.float32),
        pltpu.SemaphoreType.DMA,
        pltpu.SemaphoreType.DMA,
        pltpu.SemaphoreType.DMA,
        pltpu.SemaphoreType.DMA,
        pltpu.SemaphoreType.DMA,
    ],
)(_sc_body)


def _tc_body(s_ref, w_ref, b_ref, o_ref):
    pooled = s_ref[0] + s_ref[1]
    o_ref[...] = (
        lax.dot_general(w_ref[...], pooled, (((1,), (1,)), ((), ())),
                        preferred_element_type=jnp.float32)
        + b_ref[...]
    )


_tc_call = pl.pallas_call(
    _tc_body,
    out_shape=jax.ShapeDtypeStruct((_NCLS, _B), jnp.float32),
)


def kernel(text, offsets, emb_table, W, b):
    tab_rm = emb_table
    (s_parts,) = _sc_call(text, offsets, tab_rm)
    ot = _tc_call(s_parts, W, b.reshape(_NCLS, 1))
    return ot.T


# speedup vs baseline: 154.0951x; 1.0234x over previous
import functools

import jax
import jax.numpy as jnp
from jax import lax
from jax.experimental import pallas as pl
from jax.experimental.pallas import tpu as pltpu
from jax.experimental.pallas import tpu_sc as plsc

_VOCAB = 100000
_EMB = 64
_NCLS = 4
_B = 4096
_N = 204800

_NC = 2
_NS = 16
_NW = _NC * _NS
_CHUNK = _N // _NW
_SUB = 128
_NSUB = _CHUNK // _SUB
_BPW = _B // _NW
_BPT = _B // _NS


def _seg_body(off_h, seg_h, inv_h, off_v, seg1, inv_v, osem):
    cid = lax.axis_index("c")
    sid = lax.axis_index("s")
    wid = sid * _NC + cid
    s0 = wid * _CHUNK

    odesc = pltpu.async_copy(off_h, off_v.at[pl.ds(0, _B)], osem)

    z16 = jnp.zeros((16,), jnp.int32)

    def _zseg(k, c):
        for t in range(4):
            seg1[pl.ds(k * 64 + t * 16, 16)] = z16
        return c
    lax.fori_loop(0, _CHUNK // 64, _zseg, 0)

    odesc.wait()
    off_v[pl.ds(_B, 16)] = jnp.full((16,), _N, jnp.int32)

    ones16 = jnp.ones((16,), jnp.int32)

    def _mark(i, acc):
        a = acc
        for t in range(4):
            ov = off_v[pl.ds(i * 64 + t * 16, 16)]
            p = ov - s0
            m = (p >= 0) & (p < _CHUNK)
            plsc.store_scatter(seg1, [p], ones16, mask=m)
            a = a + jnp.where(ov < s0, 1, 0)
        return a

    acc = lax.fori_loop(0, _B // 64, _mark, jnp.zeros((16,), jnp.int32))
    base = jnp.sum(acc)

    def _cum(k, carry):
        vs = [seg1[pl.ds(k * 64 + t * 16, 16)] for t in range(4)]
        css = [jnp.cumsum(v) for v in vs]
        sums = [jnp.sum(v) for v in vs]
        c = carry
        for t in range(4):
            seg1[pl.ds(k * 64 + t * 16, 16)] = css[t] + c
            c = c + sums[t]
        return c

    lax.fori_loop(0, _CHUNK // 64, _cum, base - 1)

    wdesc = pltpu.async_copy(seg1, seg_h.at[pl.ds(s0, _CHUNK)], osem)

    bbase = wid * _BPW
    for k in range(_BPW // 16):
        a = off_v[pl.ds(bbase + k * 16, 16)]
        nx = off_v[pl.ds(bbase + k * 16 + 1, 16)]
        inv_v[pl.ds(k * 16, 16)] = 1.0 / (nx - a).astype(jnp.float32)

    wdesc.wait()
    pltpu.sync_copy(inv_v, inv_h.at[pl.ds(bbase, _BPW)])


_seg_call = functools.partial(
    pl.kernel,
    out_type=[
        jax.ShapeDtypeStruct((_N,), jnp.int32),
        jax.ShapeDtypeStruct((_B,), jnp.float32),
    ],
    mesh=plsc.VectorSubcoreMesh(core_axis_name="c", subcore_axis_name="s"),
    compiler_params=pltpu.CompilerParams(needs_layout_passes=False,
                                         use_tc_tiling_on_sc=False),
    scratch_types=[
        pltpu.VMEM((_B + 16,), jnp.int32),
        pltpu.VMEM((_CHUNK,), jnp.int32),
        pltpu.VMEM((_BPW,), jnp.float32),
        pltpu.SemaphoreType.DMA,
    ],
)(_seg_body)


def _acc_body(text_h, seg_h, inv_h, tab_h, s_h,
              text1, seg1, rows_a, rows_b, inv_v, acc_sh,
              gsa, gsb, ssa, ssb, osem):
    cid = lax.axis_index("c")
    sid = lax.axis_index("s")
    wid = sid * _NC + cid
    s0 = wid * _CHUNK
    tb = sid * _BPT

    tdesc = pltpu.async_copy(text_h.at[pl.ds(s0, _CHUNK)], text1, gsa)
    sdesc = pltpu.async_copy(seg_h.at[pl.ds(s0, _CHUNK)], seg1, gsb)
    idesc = pltpu.async_copy(inv_h.at[pl.ds(tb, _BPT)], inv_v, osem)

    zf16 = jnp.zeros((16,), jnp.float32)

    def _zrows(k, c):
        for t in range(4):
            rows_a[k, pl.ds(t * 16, 16)] = zf16
        return c
    lax.fori_loop(0, _SUB, _zrows, 0)
    pltpu.sync_copy(rows_a, acc_sh.at[pl.ds(tb, _SUB)])
    pltpu.sync_copy(rows_a, acc_sh.at[pl.ds(tb + _SUB, _SUB)])

    tdesc.wait()
    sdesc.wait()
    idesc.wait()
    plsc.subcore_barrier()

    def _fire_gather(j, buf, sem):
        return pltpu.async_copy(tab_h.at[text1.at[pl.ds(j * _SUB, _SUB)]],
                                buf, sem)

    def _fire_scatter(j, buf, sem):
        return pltpu.async_copy(buf, acc_sh.at[seg1.at[pl.ds(j * _SUB, _SUB)]],
                                sem, add=True)

    _fire_gather(0, rows_a, gsa)
    _fire_gather(1, rows_b, gsb)

    def _main(g, c):
        j0 = g * 2
        pltpu.make_async_copy(tab_h.at[text1.at[pl.ds(j0 * _SUB, _SUB)]],
                              rows_a, gsa).wait()
        sa = _fire_scatter(j0, rows_a, ssa)
        pltpu.make_async_copy(tab_h.at[text1.at[pl.ds((j0 + 1) * _SUB, _SUB)]],
                              rows_b, gsb).wait()
        sb = _fire_scatter(j0 + 1, rows_b, ssb)
        sa.wait()
        _fire_gather(j0 + 2, rows_a, gsa)
        sb.wait()
        _fire_gather(j0 + 3, rows_b, gsb)
        return c

    lax.fori_loop(0, _NSUB // 2 - 1, _main, 0)

    jl = _NSUB - 2
    pltpu.make_async_copy(tab_h.at[text1.at[pl.ds(jl * _SUB, _SUB)]],
                          rows_a, gsa).wait()
    sa = _fire_scatter(jl, rows_a, ssa)
    pltpu.make_async_copy(tab_h.at[text1.at[pl.ds((jl + 1) * _SUB, _SUB)]],
                          rows_b, gsb).wait()
    sb = _fire_scatter(jl + 1, rows_b, ssb)
    sa.wait()
    sb.wait()
    plsc.subcore_barrier()

    def _scale(buf, half):
        def _row(r, c):
            gv = plsc.load_gather(inv_v, [jnp.full((16,), half * _SUB + r,
                                                   jnp.int32)])
            for t in range(4):
                v = buf[r, pl.ds(t * 16, 16)]
                buf[r, pl.ds(t * 16, 16)] = v * gv
            return c
        lax.fori_loop(0, _SUB, _row, 0)

    pltpu.sync_copy(acc_sh.at[pl.ds(tb, _SUB)], rows_a)
    _scale(rows_a, 0)
    wa = pltpu.async_copy(rows_a, s_h.at[cid, pl.ds(tb, _SUB)], osem)
    pltpu.sync_copy(acc_sh.at[pl.ds(tb + _SUB, _SUB)], rows_b)
    _scale(rows_b, 1)
    wa.wait()
    pltpu.sync_copy(rows_b, s_h.at[cid, pl.ds(tb + _SUB, _SUB)])


_acc_call = functools.partial(
    pl.kernel,
    out_type=[
        jax.ShapeDtypeStruct((_NC, _B, _EMB), jnp.float32),
    ],
    mesh=plsc.VectorSubcoreMesh(core_axis_name="c", subcore_axis_name="s"),
    compiler_params=pltpu.CompilerParams(needs_layout_passes=False,
                                         use_tc_tiling_on_sc=False),
    scratch_types=[
        pltpu.VMEM((_CHUNK,), jnp.int32),
        pltpu.VMEM((_CHUNK,), jnp.int32),
        pltpu.VMEM((_SUB, _EMB), jnp.float32),
        pltpu.VMEM((_SUB, _EMB), jnp.float32),
        pltpu.VMEM((_BPT,), jnp.float32),
        pltpu.VMEM_SHARED((_B, _EMB), jnp.float32),
        pltpu.SemaphoreType.DMA,
        pltpu.SemaphoreType.DMA,
        pltpu.SemaphoreType.DMA,
        pltpu.SemaphoreType.DMA,
        pltpu.SemaphoreType.DMA,
    ],
)(_acc_body)


def _tc_body(s_ref, w_ref, b_ref, o_ref):
    pooled = s_ref[0] + s_ref[1]
    o_ref[...] = (
        lax.dot_general(w_ref[...], pooled, (((1,), (1,)), ((), ())),
                        preferred_element_type=jnp.float32)
        + b_ref[...]
    )


_tc_call = pl.pallas_call(
    _tc_body,
    out_shape=jax.ShapeDtypeStruct((_NCLS, _B), jnp.float32),
)


def kernel(text, offsets, emb_table, W, b):
    seg, invc = _seg_call(offsets)
    (s_parts,) = _acc_call(text, seg, invc, emb_table)
    ot = _tc_call(s_parts, W, b.reshape(_NCLS, 1))
    return ot.T


# speedup vs baseline: 170.9411x; 1.1093x over previous
import functools

import jax
import jax.numpy as jnp
from jax import lax
from jax.experimental import pallas as pl
from jax.experimental.pallas import tpu as pltpu
from jax.experimental.pallas import tpu_sc as plsc

_VOCAB = 100000
_EMB = 64
_NCLS = 4
_B = 4096
_N = 204800

_NC = 2
_NS = 16
_NW = _NC * _NS
_CHUNK = _N // _NW
_SUB = 128
_NSUB = _CHUNK // _SUB
_BPW = _B // _NW
_BPT = _B // _NS


def _seg_body(off_h, seg_h, inv_h, off_v, seg1, inv_v, osem):
    cid = lax.axis_index("c")
    sid = lax.axis_index("s")
    wid = sid * _NC + cid
    s0 = wid * _CHUNK

    odesc = pltpu.async_copy(off_h, off_v.at[pl.ds(0, _B)], osem)

    z16 = jnp.zeros((16,), jnp.int32)

    def _zseg(k, c):
        for t in range(4):
            seg1[pl.ds(k * 64 + t * 16, 16)] = z16
        return c
    lax.fori_loop(0, _CHUNK // 64, _zseg, 0)

    odesc.wait()
    off_v[pl.ds(_B, 16)] = jnp.full((16,), _N, jnp.int32)

    ones16 = jnp.ones((16,), jnp.int32)

    def _mark(i, acc):
        a = acc
        for t in range(4):
            ov = off_v[pl.ds(i * 64 + t * 16, 16)]
            p = ov - s0
            m = (p >= 0) & (p < _CHUNK)
            plsc.store_scatter(seg1, [p], ones16, mask=m)
            a = a + jnp.where(ov < s0, 1, 0)
        return a

    acc = lax.fori_loop(0, _B // 64, _mark, jnp.zeros((16,), jnp.int32))
    base = jnp.sum(acc)

    def _cum(k, carry):
        vs = [seg1[pl.ds(k * 64 + t * 16, 16)] for t in range(4)]
        css = [jnp.cumsum(v) for v in vs]
        sums = [jnp.sum(v) for v in vs]
        c = carry
        for t in range(4):
            seg1[pl.ds(k * 64 + t * 16, 16)] = css[t] + c
            c = c + sums[t]
        return c

    lax.fori_loop(0, _CHUNK // 64, _cum, base - 1)

    wdesc = pltpu.async_copy(seg1, seg_h.at[pl.ds(s0, _CHUNK)], osem)

    bbase = wid * _BPW
    for k in range(_BPW // 16):
        a = off_v[pl.ds(bbase + k * 16, 16)]
        nx = off_v[pl.ds(bbase + k * 16 + 1, 16)]
        inv_v[pl.ds(k * 16, 16)] = 1.0 / (nx - a).astype(jnp.float32)

    wdesc.wait()
    pltpu.sync_copy(inv_v, inv_h.at[pl.ds(bbase, _BPW)])


_seg_call = functools.partial(
    pl.kernel,
    out_type=[
        jax.ShapeDtypeStruct((_N,), jnp.int32),
        jax.ShapeDtypeStruct((_B,), jnp.float32),
    ],
    mesh=plsc.VectorSubcoreMesh(core_axis_name="c", subcore_axis_name="s"),
    compiler_params=pltpu.CompilerParams(needs_layout_passes=False,
                                         use_tc_tiling_on_sc=False),
    scratch_types=[
        pltpu.VMEM((_B + 16,), jnp.int32),
        pltpu.VMEM((_CHUNK,), jnp.int32),
        pltpu.VMEM((_BPW,), jnp.float32),
        pltpu.SemaphoreType.DMA,
    ],
)(_seg_body)


def _acc_body(text_h, seg_h, inv_h, tab_h, s_h,
              text1, seg1, rows_a, rows_b, rows_c, rows_d, inv_v, acc_sh,
              gsa, gsb, gsc, gsd, ssa, ssb, ssc, ssd, osem):
    cid = lax.axis_index("c")
    sid = lax.axis_index("s")
    wid = sid * _NC + cid
    s0 = wid * _CHUNK
    tb = sid * _BPT
    bufs = (rows_a, rows_b, rows_c, rows_d)
    gsems = (gsa, gsb, gsc, gsd)
    ssems = (ssa, ssb, ssc, ssd)

    tdesc = pltpu.async_copy(text_h.at[pl.ds(s0, _CHUNK)], text1, gsa)
    sdesc = pltpu.async_copy(seg_h.at[pl.ds(s0, _CHUNK)], seg1, gsb)
    idesc = pltpu.async_copy(inv_h.at[pl.ds(tb, _BPT)], inv_v, osem)

    zf16 = jnp.zeros((16,), jnp.float32)

    def _zrows(k, c):
        for t in range(4):
            rows_a[k, pl.ds(t * 16, 16)] = zf16
        return c
    lax.fori_loop(0, _SUB, _zrows, 0)
    pltpu.sync_copy(rows_a, acc_sh.at[pl.ds(tb, _SUB)])
    pltpu.sync_copy(rows_a, acc_sh.at[pl.ds(tb + _SUB, _SUB)])

    tdesc.wait()
    sdesc.wait()
    idesc.wait()
    plsc.subcore_barrier()

    def _fire_gather(j, t):
        return pltpu.async_copy(tab_h.at[text1.at[pl.ds(j * _SUB, _SUB)]],
                                bufs[t], gsems[t])

    def _wait_gather(j, t):
        pltpu.make_async_copy(tab_h.at[text1.at[pl.ds(j * _SUB, _SUB)]],
                              bufs[t], gsems[t]).wait()

    def _fire_scatter(j, t):
        return pltpu.async_copy(bufs[t],
                                acc_sh.at[seg1.at[pl.ds(j * _SUB, _SUB)]],
                                ssems[t], add=True)

    def _wait_scatter(j, t):
        pltpu.make_async_copy(bufs[t],
                              acc_sh.at[seg1.at[pl.ds(j * _SUB, _SUB)]],
                              ssems[t]).wait()

    for t in range(4):
        _fire_gather(t, t)

    def _main(g, c):
        j0 = g * 4
        for t in range(4):
            _wait_gather(j0 + t, t)
            _fire_scatter(j0 + t, t)
        for t in range(4):
            _wait_scatter(j0 + t, t)
            _fire_gather(j0 + 4 + t, t)
        return c

    lax.fori_loop(0, _NSUB // 4 - 1, _main, 0)

    jl = (_NSUB // 4 - 1) * 4
    for t in range(4):
        _wait_gather(jl + t, t)
        _fire_scatter(jl + t, t)
    for t in range(_NSUB - jl - 4):
        _wait_scatter(jl + t, t)
        _fire_gather(jl + 4 + t, t)
    for t in range(_NSUB - jl - 4, 4):
        _wait_scatter(jl + t, t)
    for t in range(_NSUB - jl - 4):
        _wait_gather(jl + 4 + t, t)
        _fire_scatter(jl + 4 + t, t)
    for t in range(_NSUB - jl - 4):
        _wait_scatter(jl + 4 + t, t)
    plsc.subcore_barrier()

    def _scale(buf, half):
        def _row(r, c):
            gv = plsc.load_gather(inv_v, [jnp.full((16,), half * _SUB + r,
                                                   jnp.int32)])
            for t in range(4):
                v = buf[r, pl.ds(t * 16, 16)]
                buf[r, pl.ds(t * 16, 16)] = v * gv
            return c
        lax.fori_loop(0, _SUB, _row, 0)

    pltpu.sync_copy(acc_sh.at[pl.ds(tb, _SUB)], rows_a)
    _scale(rows_a, 0)
    wa = pltpu.async_copy(rows_a, s_h.at[cid, pl.ds(tb, _SUB)], osem)
    pltpu.sync_copy(acc_sh.at[pl.ds(tb + _SUB, _SUB)], rows_b)
    _scale(rows_b, 1)
    wa.wait()
    pltpu.sync_copy(rows_b, s_h.at[cid, pl.ds(tb + _SUB, _SUB)])


_acc_call = functools.partial(
    pl.kernel,
    out_type=[
        jax.ShapeDtypeStruct((_NC, _B, _EMB), jnp.float32),
    ],
    mesh=plsc.VectorSubcoreMesh(core_axis_name="c", subcore_axis_name="s"),
    compiler_params=pltpu.CompilerParams(needs_layout_passes=False,
                                         use_tc_tiling_on_sc=False),
    scratch_types=[
        pltpu.VMEM((_CHUNK,), jnp.int32),
        pltpu.VMEM((_CHUNK,), jnp.int32),
        pltpu.VMEM((_SUB, _EMB), jnp.float32),
        pltpu.VMEM((_SUB, _EMB), jnp.float32),
        pltpu.VMEM((_SUB, _EMB), jnp.float32),
        pltpu.VMEM((_SUB, _EMB), jnp.float32),
        pltpu.VMEM((_BPT,), jnp.float32),
        pltpu.VMEM_SHARED((_B, _EMB), jnp.float32),
        pltpu.SemaphoreType.DMA,
        pltpu.SemaphoreType.DMA,
        pltpu.SemaphoreType.DMA,
        pltpu.SemaphoreType.DMA,
        pltpu.SemaphoreType.DMA,
        pltpu.SemaphoreType.DMA,
        pltpu.SemaphoreType.DMA,
        pltpu.SemaphoreType.DMA,
        pltpu.SemaphoreType.DMA,
    ],
)(_acc_body)


def _tc_body(s_ref, w_ref, b_ref, o_ref):
    pooled = s_ref[0] + s_ref[1]
    o_ref[...] = (
        lax.dot_general(w_ref[...], pooled, (((1,), (1,)), ((), ())),
                        preferred_element_type=jnp.float32)
        + b_ref[...]
    )


_tc_call = pl.pallas_call(
    _tc_body,
    out_shape=jax.ShapeDtypeStruct((_NCLS, _B), jnp.float32),
)


def kernel(text, offsets, emb_table, W, b):
    seg, invc = _seg_call(offsets)
    (s_parts,) = _acc_call(text, seg, invc, emb_table)
    ot = _tc_call(s_parts, W, b.reshape(_NCLS, 1))
    return ot.T


# speedup vs baseline: 171.3956x; 1.0027x over previous
import functools

import jax
import jax.numpy as jnp
from jax import lax
from jax.experimental import pallas as pl
from jax.experimental.pallas import tpu as pltpu
from jax.experimental.pallas import tpu_sc as plsc

_VOCAB = 100000
_EMB = 64
_NCLS = 4
_B = 4096
_N = 204800

_NC = 2
_NS = 16
_NW = _NC * _NS
_CHUNK = _N // _NW
_SUB = 256
_NSUB = _CHUNK // _SUB
_BPW = _B // _NW
_BPT = _B // _NS


def _seg_body(off_h, seg_h, inv_h, off_v, seg1, inv_v, osem):
    cid = lax.axis_index("c")
    sid = lax.axis_index("s")
    wid = sid * _NC + cid
    s0 = wid * _CHUNK

    odesc = pltpu.async_copy(off_h, off_v.at[pl.ds(0, _B)], osem)

    z16 = jnp.zeros((16,), jnp.int32)

    def _zseg(k, c):
        for t in range(4):
            seg1[pl.ds(k * 64 + t * 16, 16)] = z16
        return c
    lax.fori_loop(0, _CHUNK // 64, _zseg, 0)

    odesc.wait()
    off_v[pl.ds(_B, 16)] = jnp.full((16,), _N, jnp.int32)

    ones16 = jnp.ones((16,), jnp.int32)

    def _mark(i, acc):
        a = acc
        for t in range(4):
            ov = off_v[pl.ds(i * 64 + t * 16, 16)]
            p = ov - s0
            m = (p >= 0) & (p < _CHUNK)
            plsc.store_scatter(seg1, [p], ones16, mask=m)
            a = a + jnp.where(ov < s0, 1, 0)
        return a

    acc = lax.fori_loop(0, _B // 64, _mark, jnp.zeros((16,), jnp.int32))
    base = jnp.sum(acc)

    def _cum(k, carry):
        vs = [seg1[pl.ds(k * 64 + t * 16, 16)] for t in range(4)]
        css = [jnp.cumsum(v) for v in vs]
        sums = [jnp.sum(v) for v in vs]
        c = carry
        for t in range(4):
            seg1[pl.ds(k * 64 + t * 16, 16)] = css[t] + c
            c = c + sums[t]
        return c

    lax.fori_loop(0, _CHUNK // 64, _cum, base - 1)

    wdesc = pltpu.async_copy(seg1, seg_h.at[pl.ds(s0, _CHUNK)], osem)

    bbase = wid * _BPW
    for k in range(_BPW // 16):
        a = off_v[pl.ds(bbase + k * 16, 16)]
        nx = off_v[pl.ds(bbase + k * 16 + 1, 16)]
        inv_v[pl.ds(k * 16, 16)] = 1.0 / (nx - a).astype(jnp.float32)

    wdesc.wait()
    pltpu.sync_copy(inv_v, inv_h.at[pl.ds(bbase, _BPW)])


_seg_call = functools.partial(
    pl.kernel,
    out_type=[
        jax.ShapeDtypeStruct((_N,), jnp.int32),
        jax.ShapeDtypeStruct((_B,), jnp.float32),
    ],
    mesh=plsc.VectorSubcoreMesh(core_axis_name="c", subcore_axis_name="s"),
    compiler_params=pltpu.CompilerParams(needs_layout_passes=False,
                                         use_tc_tiling_on_sc=False),
    scratch_types=[
        pltpu.VMEM((_B + 16,), jnp.int32),
        pltpu.VMEM((_CHUNK,), jnp.int32),
        pltpu.VMEM((_BPW,), jnp.float32),
        pltpu.SemaphoreType.DMA,
    ],
)(_seg_body)


def _acc_body(text_h, seg_h, inv_h, tab_h, s_h,
              text1, seg1, rows_a, rows_b, rows_c, rows_d, inv_v, acc_sh,
              gsa, gsb, gsc, gsd, ssa, ssb, ssc, ssd, osem):
    cid = lax.axis_index("c")
    sid = lax.axis_index("s")
    wid = sid * _NC + cid
    s0 = wid * _CHUNK
    tb = sid * _BPT
    bufs = (rows_a, rows_b, rows_c, rows_d)
    gsems = (gsa, gsb, gsc, gsd)
    ssems = (ssa, ssb, ssc, ssd)

    tdesc = pltpu.async_copy(text_h.at[pl.ds(s0, _CHUNK)], text1, gsa)
    sdesc = pltpu.async_copy(seg_h.at[pl.ds(s0, _CHUNK)], seg1, gsb)
    idesc = pltpu.async_copy(inv_h.at[pl.ds(tb, _BPT)], inv_v, osem)

    zf16 = jnp.zeros((16,), jnp.float32)

    def _zrows(k, c):
        for t in range(4):
            rows_a[k, pl.ds(t * 16, 16)] = zf16
        return c
    lax.fori_loop(0, _SUB, _zrows, 0)
    for st in range(_BPT // _SUB):
        pltpu.sync_copy(rows_a, acc_sh.at[pl.ds(tb + st * _SUB, _SUB)])

    tdesc.wait()
    sdesc.wait()
    idesc.wait()
    plsc.subcore_barrier()

    def _fire_gather(j, t):
        return pltpu.async_copy(tab_h.at[text1.at[pl.ds(j * _SUB, _SUB)]],
                                bufs[t], gsems[t])

    def _wait_gather(j, t):
        pltpu.make_async_copy(tab_h.at[text1.at[pl.ds(j * _SUB, _SUB)]],
                              bufs[t], gsems[t]).wait()

    def _fire_scatter(j, t):
        return pltpu.async_copy(bufs[t],
                                acc_sh.at[seg1.at[pl.ds(j * _SUB, _SUB)]],
                                ssems[t], add=True)

    def _wait_scatter(j, t):
        pltpu.make_async_copy(bufs[t],
                              acc_sh.at[seg1.at[pl.ds(j * _SUB, _SUB)]],
                              ssems[t]).wait()

    for t in range(4):
        _fire_gather(t, t)

    def _main(g, c):
        j0 = g * 4
        for t in range(4):
            _wait_gather(j0 + t, t)
            _fire_scatter(j0 + t, t)
        for t in range(4):
            _wait_scatter(j0 + t, t)
            _fire_gather(j0 + 4 + t, t)
        return c

    lax.fori_loop(0, _NSUB // 4 - 1, _main, 0)

    jl = (_NSUB // 4 - 1) * 4
    for t in range(4):
        _wait_gather(jl + t, t)
        _fire_scatter(jl + t, t)
    for t in range(_NSUB - jl - 4):
        _wait_scatter(jl + t, t)
        _fire_gather(jl + 4 + t, t)
    for t in range(_NSUB - jl - 4, 4):
        _wait_scatter(jl + t, t)
    for t in range(_NSUB - jl - 4):
        _wait_gather(jl + 4 + t, t)
        _fire_scatter(jl + 4 + t, t)
    for t in range(_NSUB - jl - 4):
        _wait_scatter(jl + 4 + t, t)
    plsc.subcore_barrier()

    def _scale(buf, half):
        def _row(r, c):
            gv = plsc.load_gather(inv_v, [jnp.full((16,), half * _SUB + r,
                                                   jnp.int32)])
            for t in range(4):
                v = buf[r, pl.ds(t * 16, 16)]
                buf[r, pl.ds(t * 16, 16)] = v * gv
            return c
        lax.fori_loop(0, _SUB, _row, 0)

    wdescs = []
    for st in range(_BPT // _SUB):
        pltpu.sync_copy(acc_sh.at[pl.ds(tb + st * _SUB, _SUB)], bufs[st])
        _scale(bufs[st], st)
        wdescs.append(pltpu.async_copy(
            bufs[st], s_h.at[cid, pl.ds(tb + st * _SUB, _SUB)], osem))
    for d in wdescs:
        d.wait()


_acc_call = functools.partial(
    pl.kernel,
    out_type=[
        jax.ShapeDtypeStruct((_NC, _B, _EMB), jnp.float32),
    ],
    mesh=plsc.VectorSubcoreMesh(core_axis_name="c", subcore_axis_name="s"),
    compiler_params=pltpu.CompilerParams(needs_layout_passes=False,
                                         use_tc_tiling_on_sc=False),
    scratch_types=[
        pltpu.VMEM((_CHUNK,), jnp.int32),
        pltpu.VMEM((_CHUNK,), jnp.int32),
        pltpu.VMEM((_SUB, _EMB), jnp.float32),
        pltpu.VMEM((_SUB, _EMB), jnp.float32),
        pltpu.VMEM((_SUB, _EMB), jnp.float32),
        pltpu.VMEM((_SUB, _EMB), jnp.float32),
        pltpu.VMEM((_BPT,), jnp.float32),
        pltpu.VMEM_SHARED((_B, _EMB), jnp.float32),
        pltpu.SemaphoreType.DMA,
        pltpu.SemaphoreType.DMA,
        pltpu.SemaphoreType.DMA,
        pltpu.SemaphoreType.DMA,
        pltpu.SemaphoreType.DMA,
        pltpu.SemaphoreType.DMA,
        pltpu.SemaphoreType.DMA,
        pltpu.SemaphoreType.DMA,
        pltpu.SemaphoreType.DMA,
    ],
)(_acc_body)


def _tc_body(s_ref, w_ref, b_ref, o_ref):
    pooled = s_ref[0] + s_ref[1]
    o_ref[...] = (
        lax.dot_general(w_ref[...], pooled, (((1,), (1,)), ((), ())),
                        preferred_element_type=jnp.float32)
        + b_ref[...]
    )


_tc_call = pl.pallas_call(
    _tc_body,
    out_shape=jax.ShapeDtypeStruct((_NCLS, _B), jnp.float32),
)


def kernel(text, offsets, emb_table, W, b):
    seg, invc = _seg_call(offsets)
    (s_parts,) = _acc_call(text, seg, invc, emb_table)
    ot = _tc_call(s_parts, W, b.reshape(_NCLS, 1))
    return ot.T
